# Initial kernel scaffold; baseline (speedup 1.0000x reference)
import jax
import jax.numpy as jnp
from jax.experimental import pallas as pl


def kernel(s, nbrs_s, r, candidates, nbrs_candidates, labels, entities_emb, relations_emb):
    raise NotImplementedError("write your pallas kernel here")


# speedup vs baseline: 3.6068x; 3.6068x over previous
import functools

import jax
import jax.numpy as jnp
from jax import lax
from jax.experimental import pallas as pl
from jax.experimental.pallas import tpu as pltpu
from jax.experimental.pallas import tpu_sc as plsc

B = 4096
C = 200
CP = 208
D = 64
NC, NS, L = 2, 16, 16
NW = NC * NS
BPW = B // NW
CH = 2
NCH = BPW // CH
ROWS = CH * CP
SUB = 104
NSUB = ROWS // SUB


def _sc_kernel_body(cand_hbm, s_hbm, r_hbm, ent_hbm, rel_hbm, out_hbm,
                    sidx_v, ridx_v, q_v, r_v, cidx_v, cand_v, scores_v,
                    colbuf, sem):
    wid = lax.axis_index("s") * NC + lax.axis_index("c")
    rowbase = wid * BPW

    pltpu.sync_copy(s_hbm.at[pl.ds(rowbase, BPW)], sidx_v)
    pltpu.sync_copy(r_hbm.at[pl.ds(rowbase, BPW)], ridx_v)
    pltpu.async_copy(ent_hbm.at[sidx_v], q_v, sem).wait()
    pltpu.async_copy(rel_hbm.at[ridx_v], r_v, sem).wait()

    def qbody(i, _):
        for k in range(D // L):
            q_v[i, pl.ds(k * L, L)] = (q_v[i, pl.ds(k * L, L)]
                                       + r_v[i, pl.ds(k * L, L)])
        return 0

    lax.fori_loop(0, BPW, qbody, 0)

    lane = lax.iota(jnp.int32, L)

    def chunk_body(g, _):
        coff = (rowbase + g * CH) * CP
        pltpu.sync_copy(cand_hbm.at[pl.ds(coff, ROWS)], cidx_v)
        copies = [
            pltpu.async_copy(ent_hbm.at[cidx_v.at[pl.ds(k * SUB, SUB)]],
                             cand_v.at[pl.ds(k * SUB, SUB)], sem)
            for k in range(NSUB)
        ]
        for cp in copies:
            cp.wait()

        for row in range(CH):
            rw = g * CH + row
            q0 = q_v[rw, pl.ds(0, L)]
            q1 = q_v[rw, pl.ds(L, L)]
            q2 = q_v[rw, pl.ds(2 * L, L)]
            q3 = q_v[rw, pl.ds(3 * L, L)]

            def grp_body(grp, _, row=row, q0=q0, q1=q1, q2=q2, q3=q3):
                base = row * CP + grp * L
                for c16 in range(L):
                    fc = base + c16
                    a0 = jnp.abs(q0 - cand_v[fc, pl.ds(0, L)])
                    a1 = jnp.abs(q1 - cand_v[fc, pl.ds(L, L)])
                    a2 = jnp.abs(q2 - cand_v[fc, pl.ds(2 * L, L)])
                    a3 = jnp.abs(q3 - cand_v[fc, pl.ds(3 * L, L)])
                    acc = (a0 + a1) + (a2 + a3)
                    plsc.store_scatter(colbuf, [lane * L + c16], acc)
                sv = colbuf[pl.ds(0, L)]
                for l in range(1, L):
                    sv = sv + colbuf[pl.ds(l * L, L)]
                scores_v[pl.ds(base, L)] = sv
                return 0

            lax.fori_loop(0, CP // L, grp_body, 0)

        pltpu.sync_copy(scores_v, out_hbm.at[pl.ds(coff, ROWS)])
        return 0

    lax.fori_loop(0, NCH, chunk_body, 0)


@jax.jit
def _transe_scores(cand_flat, s, r, entities_emb, relations_emb):
    mesh = plsc.VectorSubcoreMesh(core_axis_name="c", subcore_axis_name="s")
    kfn = pl.kernel(
        _sc_kernel_body,
        out_type=jax.ShapeDtypeStruct((B * CP,), jnp.float32),
        mesh=mesh,
        compiler_params=pltpu.CompilerParams(needs_layout_passes=False,
                                             use_tc_tiling_on_sc=False),
        scratch_types=[
            pltpu.VMEM((BPW,), jnp.int32),
            pltpu.VMEM((BPW,), jnp.int32),
            pltpu.VMEM((BPW, D), jnp.float32),
            pltpu.VMEM((BPW, D), jnp.float32),
            pltpu.VMEM((ROWS,), jnp.int32),
            pltpu.VMEM((ROWS, D), jnp.float32),
            pltpu.VMEM((ROWS,), jnp.float32),
            pltpu.VMEM((L * L,), jnp.float32),
            pltpu.SemaphoreType.DMA,
        ],
    )
    return kfn(cand_flat, s, r, entities_emb, relations_emb)


def kernel(s, nbrs_s, r, candidates, nbrs_candidates, labels,
           entities_emb, relations_emb):
    del nbrs_s, nbrs_candidates, labels
    cand_p = jnp.pad(candidates.astype(jnp.int32), ((0, 0), (0, CP - C)))
    cand_flat = cand_p.reshape(-1)
    out = _transe_scores(cand_flat, s.astype(jnp.int32), r.astype(jnp.int32),
                         entities_emb, relations_emb)
    return out.reshape(B, CP)[:, :C]
